# Initial kernel scaffold; baseline (speedup 1.0000x reference)
import functools

import jax
import jax.numpy as jnp
from jax import lax
from jax.experimental import pallas as pl
from jax.experimental.pallas import tpu as pltpu
from jax.experimental.pallas import tpu_sc as plsc


@functools.cache
def _build(n_rows, dim, chunk):
    mesh = plsc.VectorSubcoreMesh(core_axis_name="c", subcore_axis_name="s")
    nc = mesh.num_cores
    ns = mesh.num_subcores
    n_workers = nc * ns
    rows_per_w = n_rows // n_workers
    n_chunks = rows_per_w // chunk

    def body(idx_hbm, table_hbm, out_hbm, idx_v, rows_v, sem0, sem1):
        wid = lax.axis_index("s") * nc + lax.axis_index("c")
        base = wid * rows_per_w
        pltpu.sync_copy(idx_hbm.at[pl.ds(base, rows_per_w)], idx_v)

        sems = (sem0, sem1)

        def start(ci):
            buf = ci % 2
            return pltpu.async_copy(
                table_hbm.at[idx_v.at[pl.ds(ci * chunk, chunk)]],
                rows_v.at[buf],
                sems[buf],
            )

        handles = [None, None]
        handles[0] = start(0)
        for ci in range(n_chunks):
            if ci + 1 < n_chunks:
                handles[(ci + 1) % 2] = start(ci + 1)
            handles[ci % 2].wait()
            pltpu.sync_copy(
                rows_v.at[ci % 2],
                out_hbm.at[pl.ds(base + ci * chunk, chunk)],
            )

    return pl.kernel(
        body,
        out_type=jax.ShapeDtypeStruct((n_rows, dim), jnp.float32),
        mesh=mesh,
        scratch_types=[
            pltpu.VMEM((rows_per_w,), jnp.int32),
            pltpu.VMEM((2, chunk, dim), jnp.float32),
            pltpu.SemaphoreType.DMA,
            pltpu.SemaphoreType.DMA,
        ],
    )


def kernel(indices, table):
    b, h = indices.shape
    _, d = table.shape
    n_rows = b * h
    idx_flat = indices.reshape(n_rows).astype(jnp.int32)
    out = _build(n_rows, d, 1024)(idx_flat, table)
    return out.reshape(b, h, d)


# speedup vs baseline: 1.1132x; 1.1132x over previous
import functools

import jax
import jax.numpy as jnp
from jax import lax
from jax.experimental import pallas as pl
from jax.experimental.pallas import tpu as pltpu
from jax.experimental.pallas import tpu_sc as plsc


@functools.cache
def _build(n_rows, dim, chunk):
    mesh = plsc.VectorSubcoreMesh(core_axis_name="c", subcore_axis_name="s")
    nc = mesh.num_cores
    ns = mesh.num_subcores
    n_workers = nc * ns
    rows_per_w = n_rows // n_workers
    n_chunks = rows_per_w // chunk

    def body(idx_hbm, table_hbm, out_hbm, idx_v, rows_v, sem0, sem1):
        wid = lax.axis_index("s") * nc + lax.axis_index("c")
        base = wid * rows_per_w
        pltpu.sync_copy(idx_hbm.at[pl.ds(base, rows_per_w)], idx_v)

        sems = (sem0, sem1)

        def start(ci):
            buf = ci % 2
            return pltpu.async_copy(
                table_hbm.at[idx_v.at[pl.ds(ci * chunk, chunk)]],
                rows_v.at[buf],
                sems[buf],
            )

        handles = [None, None]
        handles[0] = start(0)
        for ci in range(n_chunks):
            if ci + 1 < n_chunks:
                handles[(ci + 1) % 2] = start(ci + 1)
            handles[ci % 2].wait()
            pltpu.sync_copy(
                rows_v.at[ci % 2],
                out_hbm.at[pl.ds(base + ci * chunk, chunk)],
            )

    return pl.kernel(
        body,
        out_type=jax.ShapeDtypeStruct((n_rows, dim), jnp.float32),
        mesh=mesh,
        scratch_types=[
            pltpu.VMEM((rows_per_w,), jnp.int32),
            pltpu.VMEM((2, chunk, dim), jnp.float32),
            pltpu.SemaphoreType.DMA,
            pltpu.SemaphoreType.DMA,
        ],
        compiler_params=pltpu.CompilerParams(use_tc_tiling_on_sc=False),
    )


def kernel(indices, table):
    b, h = indices.shape
    _, d = table.shape
    n_rows = b * h
    idx_flat = indices.reshape(n_rows).astype(jnp.int32)
    out = _build(n_rows, d, 1024)(idx_flat, table)
    return out.reshape(b, h, d)
